# Initial kernel scaffold; baseline (speedup 1.0000x reference)
import jax
import jax.numpy as jnp
from jax.experimental import pallas as pl
from jax.experimental.pallas import tpu as pltpu

D, H, NQ, NT, NL, NR, MAX_AGENT = 1024, 256, 8192, 16, 8, 8, 2
BM = 512

_LANCZOS_C0 = 0.99999999999980993
_LANCZOS = (
    676.5203681218851, -1259.1392167224028, 771.32342877765313,
    -176.61502916214059, 12.507343278686905, -0.13857109526572012,
    9.9843695780195716e-6, 1.5056327351493116e-7,
)
_HALF_LOG_2PI = 0.9189385332046727
_LN2 = 0.6931471805599453


def _l2n(x):
    n = jnp.sqrt(jnp.sum(x * x, axis=1, keepdims=True))
    return x / jnp.maximum(n, 1e-12)


def _lgamma(x):
    z = x - 1.0
    s = jnp.full_like(z, _LANCZOS_C0)
    for i, c in enumerate(_LANCZOS):
        s = s + c / (z + (i + 1.0))
    t = z + 7.5
    return _HALF_LOG_2PI + (z + 0.5) * jnp.log(t) - t + jnp.log(s)


def _dot(a, b):
    return jnp.dot(a, b, preferred_element_type=jnp.float32)


def _dot_t(a, b):
    return jax.lax.dot_general(a, b, (((1,), (1,)), ((), ())),
                               preferred_element_type=jnp.float32)


def _softmax(x):
    m = jnp.max(x, axis=1, keepdims=True)
    e = jnp.exp(x - m)
    return e / jnp.sum(e, axis=1, keepdims=True)


def _first_true_idx(mask, iota_f, width):
    cand = jnp.where(mask, iota_f, float(width))
    idx = jnp.min(cand, axis=1, keepdims=True)
    return jnp.where(idx == float(width), 0.0, idx)


def _router_kernel(q_ref, wcat_ref, tasks_ref, llms_ref, reas_ref,
                   wtask_ref, wllm_ref, wr_ref, wdp_ref, bias_ref, rnd_ref,
                   ts_ref, sel_ref, lgp_ref, rsel_ref, rlp_ref,
                   task_tab, llm_tab, re_tab):
    @pl.when(pl.program_id(0) == 0)
    def _prologue():
        a = _dot(tasks_ref[...], wtask_ref[...])
        task_tab[:, 0:H] = _l2n(a[:, 0:H] + bias_ref[3:4, :])
        task_tab[:, H:3 * H] = a[:, H:3 * H]
        b = _dot(llms_ref[...], wllm_ref[...])
        llm_tab[:, 0:H] = _l2n(b[:, 0:H] + bias_ref[4:5, :])
        llm_tab[:, H:2 * H] = b[:, H:2 * H]
        re_tab[...] = _l2n(_dot(reas_ref[...], wr_ref[...]) + bias_ref[5:6, :])

    p = _dot(q_ref[...], wcat_ref[...])

    qe = _l2n(p[:, 0:H] + bias_ref[0:1, :])
    ts = _dot_t(qe, task_tab[:, 0:H])
    ts_ref[...] = ts
    iota_t = jax.lax.broadcasted_iota(jnp.int32, (BM, NT), 1).astype(jnp.float32)
    mx = jnp.max(ts, axis=1, keepdims=True)
    sel_task = _first_true_idx(ts == mx, iota_t, NT)
    onehot_t = (iota_t == sel_task).astype(jnp.float32)

    qt = _l2n(p[:, H:2 * H] + _dot(onehot_t, task_tab[:, H:2 * H])
              + bias_ref[1:2, :])
    dlogit = _dot(qt, wdp_ref[...])[:, 0:1] + bias_ref[6:7, 0:1]
    diff = jax.nn.sigmoid(dlogit)
    lnf = diff * float(MAX_AGENT)
    lni = jnp.clip(jnp.round(lnf), 1.0, float(MAX_AGENT))

    logits = _dot_t(qt, llm_tab[:, 0:H])
    scores = _softmax(logits)
    iota_l = jax.lax.broadcasted_iota(jnp.int32, (BM, NL), 1).astype(jnp.float32)
    tri = (jax.lax.broadcasted_iota(jnp.int32, (NL, NL), 0)
           <= jax.lax.broadcasted_iota(jnp.int32, (NL, NL), 1)).astype(jnp.float32)
    sc = _dot(scores, tri)

    selected = jnp.zeros((BM, NL), jnp.float32)
    for i in range(1, MAX_AGENT + 1):
        mask = (lni >= float(i)).astype(jnp.float32)
        rnd = rnd_ref[:, i - 1:i]
        idx = _first_true_idx(sc > rnd, iota_l, NL)
        selected = selected + mask * (iota_l == idx).astype(jnp.float32)
    sel_ref[...] = selected

    lg_sel = jnp.where(selected > 1.5, _LN2, 0.0)
    lgp_ref[...] = (_lgamma(lnf + 1.0)
                    - jnp.sum(lg_sel, axis=1, keepdims=True)
                    + jnp.sum(selected * jnp.log(scores), axis=1, keepdims=True))

    qtl = _l2n(p[:, 2 * H:3 * H] + _dot(onehot_t, task_tab[:, 2 * H:3 * H])
               + _dot(selected, llm_tab[:, H:2 * H]) + bias_ref[2:3, :])
    rlogits = _dot_t(qtl, re_tab[...])
    rscores = _softmax(rlogits)
    rsc = _dot(rscores, tri)
    rnd2 = rnd_ref[:, 2:3]
    ridx = _first_true_idx(rsc > rnd2, iota_l, NR)
    rsel_ref[...] = ridx.astype(jnp.int32)
    oh_r = (iota_l == ridx).astype(jnp.float32)
    rlp_ref[...] = jnp.log(jnp.sum(oh_r * rscores, axis=1, keepdims=True))


def kernel(queries, tasks, llms, reasonings, Wq, bq, Wt, bt, Wqt, bqt,
           Wl, bl, Wd, bd, Wqtl, bqtl, Wr, br):
    f32 = jnp.float32
    wcat = jnp.concatenate([Wq, Wqt[:D], Wqtl[:D]], axis=1)
    wtask = jnp.concatenate([Wt, Wqt[D:2 * D], Wqtl[D:2 * D]], axis=1)
    wllm = jnp.concatenate([Wl, Wqtl[2 * D:3 * D]], axis=1)
    wdp = jnp.pad(Wd, ((0, 0), (0, 7)))
    bias = jnp.stack([bq, bqt, bqtl, bt, bl, br,
                      jnp.broadcast_to(bd, (H,)), jnp.zeros((H,), f32)], axis=0)

    rkey = jax.random.key(42)
    rnd_cols = [jax.random.uniform(jax.random.fold_in(rkey, i), (NQ, 1))
                for i in range(1, MAX_AGENT + 1)]
    rnd_cols.append(jax.random.uniform(jax.random.fold_in(rkey, 999), (NQ, 1)))
    rnd_cols.append(jnp.zeros((NQ, 1), f32))
    rnd = jnp.concatenate(rnd_cols, axis=1)

    grid = (NQ // BM,)
    row_spec = lambda w: pl.BlockSpec((BM, w), lambda i: (i, 0))
    rep_spec = lambda r, w: pl.BlockSpec((r, w), lambda i: (0, 0))

    out = pl.pallas_call(
        _router_kernel,
        grid=grid,
        in_specs=[
            row_spec(D),
            rep_spec(D, 3 * H),
            rep_spec(NT, D),
            rep_spec(NL, D),
            rep_spec(NR, D),
            rep_spec(D, 3 * H),
            rep_spec(D, 2 * H),
            rep_spec(D, H),
            rep_spec(H, 8),
            rep_spec(8, H),
            row_spec(4),
        ],
        out_specs=[
            row_spec(NT), row_spec(NL), row_spec(1), row_spec(1), row_spec(1),
        ],
        out_shape=[
            jax.ShapeDtypeStruct((NQ, NT), f32),
            jax.ShapeDtypeStruct((NQ, NL), f32),
            jax.ShapeDtypeStruct((NQ, 1), f32),
            jax.ShapeDtypeStruct((NQ, 1), jnp.int32),
            jax.ShapeDtypeStruct((NQ, 1), f32),
        ],
        scratch_shapes=[
            pltpu.VMEM((NT, 3 * H), f32),
            pltpu.VMEM((NL, 2 * H), f32),
            pltpu.VMEM((NR, H), f32),
        ],
    )(queries, wcat, tasks, llms, reasonings, wtask, wllm, Wr, wdp,
      bias, rnd)

    task_scores, selected_llm, llm_log_probs, r_sel, r_log_probs = out
    return (task_scores, selected_llm, llm_log_probs,
            r_sel.reshape(NQ), r_log_probs)


# speedup vs baseline: 1.8382x; 1.8382x over previous
import jax
import jax.numpy as jnp
from jax.experimental import pallas as pl
from jax.experimental.pallas import tpu as pltpu

D, H, NQ, NT, NL, NR, MAX_AGENT = 1024, 256, 8192, 16, 8, 8, 2
BM = 512

_LANCZOS_C0 = 0.99999999999980993
_LANCZOS = (
    676.5203681218851, -1259.1392167224028, 771.32342877765313,
    -176.61502916214059, 12.507343278686905, -0.13857109526572012,
    9.9843695780195716e-6, 1.5056327351493116e-7,
)
_HALF_LOG_2PI = 0.9189385332046727
_LN2 = 0.6931471805599453

_BF = jnp.bfloat16
_HP = jax.lax.Precision.HIGHEST


def _l2n(x):
    n = jnp.sqrt(jnp.sum(x * x, axis=1, keepdims=True))
    return x / jnp.maximum(n, 1e-12)


def _lgamma(x):
    z = x - 1.0
    s = jnp.full_like(z, _LANCZOS_C0)
    for i, c in enumerate(_LANCZOS):
        s = s + c / (z + (i + 1.0))
    t = z + 7.5
    return _HALF_LOG_2PI + (z + 0.5) * jnp.log(t) - t + jnp.log(s)


def _dot(a, b):
    return jnp.dot(a.astype(_BF), b.astype(_BF),
                   preferred_element_type=jnp.float32)


def _dot_t(a, b):
    return jax.lax.dot_general(a.astype(_BF), b.astype(_BF),
                               (((1,), (1,)), ((), ())),
                               preferred_element_type=jnp.float32)


def _gather_dot(onehot, table):
    return jnp.dot(onehot, table, preferred_element_type=jnp.float32,
                   precision=_HP)


def _softmax(x):
    m = jnp.max(x, axis=1, keepdims=True)
    e = jnp.exp(x - m)
    return e / jnp.sum(e, axis=1, keepdims=True)


def _first_true_idx(mask, iota_f, width):
    cand = jnp.where(mask, iota_f, float(width))
    idx = jnp.min(cand, axis=1, keepdims=True)
    return jnp.where(idx == float(width), 0.0, idx)


def _router_kernel(q_ref, wcat_ref, tasks_ref, llms_ref, reas_ref,
                   wtask_ref, wllm_ref, wr_ref, wdp_ref, bias_ref, rnd_ref,
                   ts_ref, sel_ref, lgp_ref, rsel_ref, rlp_ref,
                   te_tab, tq_tab, le_tab, re_tab, ptab, pairs_bf):
    @pl.when(pl.program_id(0) == 0)
    def _prologue():
        a = _dot(tasks_ref[...], wtask_ref[...])
        te_tab[...] = _l2n(a[:, 0:H] + bias_ref[3:4, :])
        tq_tab[...] = a[:, H:3 * H]
        b = _dot(llms_ref[...], wllm_ref[...])
        le_tab[...] = _l2n(b[:, 0:H] + bias_ref[4:5, :])[0:NL, :]
        re_tab[...] = (_l2n(_dot(reas_ref[...], wr_ref[...])
                            + bias_ref[5:6, :]))[0:NR, :]
        w3 = wllm_ref[:, H:2 * H]
        lf = llms_ref[...].astype(jnp.float32)
        for aa in range(NL):
            pairs_bf[8 * aa:8 * aa + NL, :] = (
                (lf[aa:aa + 1, :] + lf[0:NL, :]).astype(_BF))
        ptab[0:NL, :] = _dot(llms_ref[...], w3)[0:NL, :]
        ptab[NL:NL + NL * NL, :] = jnp.dot(
            pairs_bf[...], w3, preferred_element_type=jnp.float32)

    p = _dot(q_ref[...], wcat_ref[...])

    qe = _l2n(p[:, 0:H] + bias_ref[0:1, :])
    ts = _dot_t(qe, te_tab[...])
    ts_ref[...] = ts
    iota_t = jax.lax.broadcasted_iota(jnp.int32, (BM, NT), 1).astype(jnp.float32)
    mx = jnp.max(ts, axis=1, keepdims=True)
    sel_task = _first_true_idx(ts == mx, iota_t, NT)
    onehot_t = (iota_t == sel_task).astype(jnp.float32)

    qt = _l2n(p[:, H:2 * H] + _gather_dot(onehot_t, tq_tab[:, 0:H])
              + bias_ref[1:2, :])
    dlogit = _dot(qt, wdp_ref[...])[:, 0:1] + bias_ref[6:7, 0:1]
    diff = jax.nn.sigmoid(dlogit)
    lnf = diff * float(MAX_AGENT)
    lni = jnp.clip(jnp.round(lnf), 1.0, float(MAX_AGENT))

    logits = _dot_t(qt, le_tab[...])
    scores = _softmax(logits)
    iota_l = jax.lax.broadcasted_iota(jnp.int32, (BM, NL), 1).astype(jnp.float32)
    tri = (jax.lax.broadcasted_iota(jnp.int32, (NL, NL), 0)
           <= jax.lax.broadcasted_iota(jnp.int32, (NL, NL), 1)).astype(jnp.float32)
    sc = jnp.dot(scores, tri, preferred_element_type=jnp.float32,
                 precision=_HP)

    idxs = []
    selected = jnp.zeros((BM, NL), jnp.float32)
    for i in range(1, MAX_AGENT + 1):
        mask = (lni >= float(i)).astype(jnp.float32)
        rnd = rnd_ref[:, i - 1:i]
        idx = _first_true_idx(sc > rnd, iota_l, NL)
        idxs.append(idx)
        selected = selected + mask * (iota_l == idx).astype(jnp.float32)
    sel_ref[...] = selected

    lg_sel = jnp.where(selected > 1.5, _LN2, 0.0)
    lgp_ref[...] = (_lgamma(lnf + 1.0)
                    - jnp.sum(lg_sel, axis=1, keepdims=True)
                    + jnp.sum(selected * jnp.log(scores), axis=1, keepdims=True))

    sel72 = jnp.where(lni < 1.5, idxs[0], 8.0 + 8.0 * idxs[0] + idxs[1])
    iota_p = jax.lax.broadcasted_iota(jnp.int32, (BM, NL + NL * NL), 1
                                      ).astype(jnp.float32)
    onehot_p = (iota_p == sel72).astype(jnp.float32)
    qtl = _l2n(p[:, 2 * H:3 * H] + _gather_dot(onehot_t, tq_tab[:, H:2 * H])
               + _gather_dot(onehot_p, ptab[...]) + bias_ref[2:3, :])
    rlogits = _dot_t(qtl, re_tab[...])
    rscores = _softmax(rlogits)
    rsc = jnp.dot(rscores, tri, preferred_element_type=jnp.float32,
                  precision=_HP)
    rnd2 = rnd_ref[:, 2:3]
    ridx = _first_true_idx(rsc > rnd2, iota_l, NR)
    rsel_ref[...] = ridx.astype(jnp.int32)
    oh_r = (iota_l == ridx).astype(jnp.float32)
    rlp_ref[...] = jnp.log(jnp.sum(oh_r * rscores, axis=1, keepdims=True))


def kernel(queries, tasks, llms, reasonings, Wq, bq, Wt, bt, Wqt, bqt,
           Wl, bl, Wd, bd, Wqtl, bqtl, Wr, br):
    f32 = jnp.float32
    q_bf = queries.astype(_BF)
    wcat = jnp.concatenate([Wq, Wqt[:D], Wqtl[:D]], axis=1).astype(_BF)
    wtask = jnp.concatenate([Wt, Wqt[D:2 * D], Wqtl[D:2 * D]], axis=1).astype(_BF)
    wllm = jnp.concatenate([Wl, Wqtl[2 * D:3 * D]], axis=1).astype(_BF)
    wr_bf = Wr.astype(_BF)
    tasks_bf = tasks.astype(_BF)
    llms_bf = jnp.pad(llms, ((0, 8), (0, 0))).astype(_BF)
    reas_bf = jnp.pad(reasonings, ((0, 8), (0, 0))).astype(_BF)
    wdp = jnp.pad(Wd, ((0, 0), (0, 7))).astype(_BF)
    bias = jnp.stack([bq, bqt, bqtl, bt, bl, br,
                      jnp.broadcast_to(bd, (H,)), jnp.zeros((H,), f32)], axis=0)

    rkey = jax.random.key(42)
    rnd_cols = [jax.random.uniform(jax.random.fold_in(rkey, i), (NQ, 1))
                for i in range(1, MAX_AGENT + 1)]
    rnd_cols.append(jax.random.uniform(jax.random.fold_in(rkey, 999), (NQ, 1)))
    rnd_cols.append(jnp.zeros((NQ, 1), f32))
    rnd = jnp.concatenate(rnd_cols, axis=1)

    grid = (NQ // BM,)
    row_spec = lambda w: pl.BlockSpec((BM, w), lambda i: (i, 0))
    rep_spec = lambda r, w: pl.BlockSpec((r, w), lambda i: (0, 0))

    out = pl.pallas_call(
        _router_kernel,
        grid=grid,
        in_specs=[
            row_spec(D),
            rep_spec(D, 3 * H),
            rep_spec(NT, D),
            rep_spec(16, D),
            rep_spec(16, D),
            rep_spec(D, 3 * H),
            rep_spec(D, 2 * H),
            rep_spec(D, H),
            rep_spec(H, 8),
            rep_spec(8, H),
            row_spec(4),
        ],
        out_specs=[
            row_spec(NT), row_spec(NL), row_spec(1), row_spec(1), row_spec(1),
        ],
        out_shape=[
            jax.ShapeDtypeStruct((NQ, NT), f32),
            jax.ShapeDtypeStruct((NQ, NL), f32),
            jax.ShapeDtypeStruct((NQ, 1), f32),
            jax.ShapeDtypeStruct((NQ, 1), jnp.int32),
            jax.ShapeDtypeStruct((NQ, 1), f32),
        ],
        scratch_shapes=[
            pltpu.VMEM((NT, H), f32),
            pltpu.VMEM((NT, 2 * H), f32),
            pltpu.VMEM((NL, H), f32),
            pltpu.VMEM((NR, H), f32),
            pltpu.VMEM((NL + NL * NL, H), f32),
            pltpu.VMEM((NL * NL, D), _BF),
        ],
    )(q_bf, wcat, tasks_bf, llms_bf, reas_bf, wtask, wllm, wr_bf, wdp,
      bias, rnd)

    task_scores, selected_llm, llm_log_probs, r_sel, r_log_probs = out
    return (task_scores, selected_llm, llm_log_probs,
            r_sel.reshape(NQ), r_log_probs)


# speedup vs baseline: 2.5442x; 1.3841x over previous
import jax
import jax.numpy as jnp
from jax.experimental import pallas as pl
from jax.experimental.pallas import tpu as pltpu

D, H, NQ, NT, NL, NR, MAX_AGENT = 1024, 256, 8192, 16, 8, 8, 2
BM = 512
NP = NL + NL * NL
NPP = 80

_LANCZOS_C0 = 0.99999999999980993
_LANCZOS = (
    676.5203681218851, -1259.1392167224028, 771.32342877765313,
    -176.61502916214059, 12.507343278686905, -0.13857109526572012,
    9.9843695780195716e-6, 1.5056327351493116e-7,
)
_HALF_LOG_2PI = 0.9189385332046727
_LN2 = 0.6931471805599453

_BF = jnp.bfloat16
_HP = jax.lax.Precision.HIGHEST


def _l2n(x):
    n = jnp.sqrt(jnp.sum(x * x, axis=1, keepdims=True))
    return x / jnp.maximum(n, 1e-12)


def _lgamma(x):
    z = x - 1.0
    s = jnp.full_like(z, _LANCZOS_C0)
    for i, c in enumerate(_LANCZOS):
        s = s + c / (z + (i + 1.0))
    t = z + 7.5
    return _HALF_LOG_2PI + (z + 0.5) * jnp.log(t) - t + jnp.log(s)


def _dot(a, b):
    return jnp.dot(a.astype(_BF), b.astype(_BF),
                   preferred_element_type=jnp.float32)


def _dot_t(a, b):
    return jax.lax.dot_general(a.astype(_BF), b.astype(_BF),
                               (((1,), (1,)), ((), ())),
                               preferred_element_type=jnp.float32)


def _softmax(x):
    m = jnp.max(x, axis=1, keepdims=True)
    e = jnp.exp(x - m)
    return e / jnp.sum(e, axis=1, keepdims=True)


def _first_true_idx(mask, iota_f, width):
    cand = jnp.where(mask, iota_f, float(width))
    idx = jnp.min(cand, axis=1, keepdims=True)
    return jnp.where(idx == float(width), 0.0, idx)


def _split3(x):
    hi = x.astype(_BF)
    r = x - hi.astype(jnp.float32)
    mid = r.astype(_BF)
    lo = (r - mid.astype(jnp.float32)).astype(_BF)
    return hi, mid, lo


def _iota_f(shape, dim):
    return jax.lax.broadcasted_iota(jnp.int32, shape, dim).astype(jnp.float32)


def _onehot3(sel, width, stride):
    i3 = _iota_f((BM, 3 * stride), 1)
    oh = ((i3 == sel).astype(jnp.float32)
          + (i3 == sel + float(stride)).astype(jnp.float32)
          + (i3 == sel + 2.0 * float(stride)).astype(jnp.float32))
    return oh.astype(_BF)


def _router_kernel(q_ref, wq_ref, wqt_a_ref, wqt_b_ref, wqtl_a_ref,
                   wqtl_b_ref, wqtl_c_ref, wt_ref, wl_ref, wr_ref,
                   tasks_ref, lr_ref, wdp_ref, bias_ref, rnd_ref,
                   ts_ref, sel_ref, lgp_ref, rsel_ref, rlp_ref,
                   wcat_bf, te_tab, le_tab, re_tab, tq3, ptab3, pairs_bf):
    @pl.when(pl.program_id(0) == 0)
    def _prologue():
        wcat_bf[:, 0:H] = wq_ref[...].astype(_BF)
        wcat_bf[:, H:2 * H] = wqt_a_ref[...].astype(_BF)
        wcat_bf[:, 2 * H:3 * H] = wqtl_a_ref[...].astype(_BF)
        tasks_bf = tasks_ref[...].astype(_BF)
        te_tab[...] = _l2n(
            jnp.dot(tasks_bf, wt_ref[...].astype(_BF),
                    preferred_element_type=jnp.float32) + bias_ref[3:4, :])
        xqt = jnp.dot(tasks_bf, wqt_b_ref[...].astype(_BF),
                      preferred_element_type=jnp.float32)
        xqtl = jnp.dot(tasks_bf, wqtl_b_ref[...].astype(_BF),
                       preferred_element_type=jnp.float32)
        h, m, l = _split3(jnp.concatenate([xqt, xqtl], axis=1))
        tq3[0:NT, :] = h
        tq3[NT:2 * NT, :] = m
        tq3[2 * NT:3 * NT, :] = l
        lr_bf = lr_ref[...].astype(_BF)
        cl = jnp.dot(lr_bf, wl_ref[...].astype(_BF),
                     preferred_element_type=jnp.float32)
        le_tab[...] = _l2n(cl[0:NL, :] + bias_ref[4:5, :])
        cr = jnp.dot(lr_bf, wr_ref[...].astype(_BF),
                     preferred_element_type=jnp.float32)
        re_tab[...] = _l2n(cr[NL:NL + NR, :] + bias_ref[5:6, :])
        w3_bf = wqtl_c_ref[...].astype(_BF)
        t1 = jnp.dot(lr_bf, w3_bf, preferred_element_type=jnp.float32)[0:NL, :]
        lfr = lr_ref[0:NL, :].astype(_BF).astype(jnp.float32)
        for a in range(NL):
            pairs_bf[NL * a:NL * a + NL, :] = (
                (lfr[a:a + 1, :] + lfr).astype(_BF))
        pp = jnp.dot(pairs_bf[...], w3_bf, preferred_element_type=jnp.float32)
        h, m, l = _split3(t1)
        ptab3[0:NL, :] = h
        ptab3[NPP:NPP + NL, :] = m
        ptab3[2 * NPP:2 * NPP + NL, :] = l
        h, m, l = _split3(pp)
        ptab3[NL:NP, :] = h
        ptab3[NPP + NL:NPP + NP, :] = m
        ptab3[2 * NPP + NL:2 * NPP + NP, :] = l
        zpad = jnp.zeros((NPP - NP, H), _BF)
        ptab3[NP:NPP, :] = zpad
        ptab3[NPP + NP:2 * NPP, :] = zpad
        ptab3[2 * NPP + NP:3 * NPP, :] = zpad

    q_bf = q_ref[...].astype(_BF)
    p = jnp.dot(q_bf, wcat_bf[...], preferred_element_type=jnp.float32)

    qe = _l2n(p[:, 0:H] + bias_ref[0:1, :])
    ts = _dot_t(qe, te_tab[...])
    ts_ref[...] = ts
    iota_t = _iota_f((BM, NT), 1)
    mx = jnp.max(ts, axis=1, keepdims=True)
    sel_task = _first_true_idx(ts == mx, iota_t, NT)

    g = jnp.dot(_onehot3(sel_task, NT, NT), tq3[...],
                preferred_element_type=jnp.float32)

    qt = _l2n(p[:, H:2 * H] + g[:, 0:H] + bias_ref[1:2, :])
    dlogit = _dot(qt, wdp_ref[...])[:, 0:1] + bias_ref[6:7, 0:1]
    diff = jax.nn.sigmoid(dlogit)
    lnf = diff * float(MAX_AGENT)
    lni = jnp.clip(jnp.round(lnf), 1.0, float(MAX_AGENT))

    logits = _dot_t(qt, le_tab[...])
    scores = _softmax(logits)
    iota_l = _iota_f((BM, NL), 1)
    tri = (jax.lax.broadcasted_iota(jnp.int32, (NL, NL), 0)
           <= jax.lax.broadcasted_iota(jnp.int32, (NL, NL), 1)).astype(jnp.float32)
    sc = jnp.dot(scores, tri, preferred_element_type=jnp.float32,
                 precision=_HP)

    idxs = []
    selected = jnp.zeros((BM, NL), jnp.float32)
    for i in range(1, MAX_AGENT + 1):
        mask = (lni >= float(i)).astype(jnp.float32)
        rnd = rnd_ref[:, i - 1:i]
        idx = _first_true_idx(sc > rnd, iota_l, NL)
        idxs.append(idx)
        selected = selected + mask * (iota_l == idx).astype(jnp.float32)
    sel_ref[...] = selected

    lg_sel = jnp.where(selected > 1.5, _LN2, 0.0)
    lgp_ref[...] = (_lgamma(lnf + 1.0)
                    - jnp.sum(lg_sel, axis=1, keepdims=True)
                    + jnp.sum(selected * jnp.log(scores), axis=1, keepdims=True))

    sel72 = jnp.where(lni < 1.5, idxs[0], 8.0 + 8.0 * idxs[0] + idxs[1])
    gp = jnp.dot(_onehot3(sel72, NP, NPP), ptab3[...],
                 preferred_element_type=jnp.float32)
    qtl = _l2n(p[:, 2 * H:3 * H] + g[:, H:2 * H] + gp + bias_ref[2:3, :])
    rlogits = _dot_t(qtl, re_tab[...])
    rscores = _softmax(rlogits)
    rsc = jnp.dot(rscores, tri, preferred_element_type=jnp.float32,
                  precision=_HP)
    rnd2 = rnd_ref[:, 2:3]
    ridx = _first_true_idx(rsc > rnd2, iota_l, NR)
    rsel_ref[...] = ridx.astype(jnp.int32)
    oh_r = (iota_l == ridx).astype(jnp.float32)
    rlp_ref[...] = jnp.log(jnp.sum(oh_r * rscores, axis=1, keepdims=True))


def kernel(queries, tasks, llms, reasonings, Wq, bq, Wt, bt, Wqt, bqt,
           Wl, bl, Wd, bd, Wqtl, bqtl, Wr, br):
    f32 = jnp.float32
    lr = jnp.concatenate([llms, reasonings], axis=0)
    wdp = jnp.pad(Wd, ((0, 0), (0, 7))).astype(_BF)
    bias = jnp.stack([bq, bqt, bqtl, bt, bl, br,
                      jnp.broadcast_to(bd, (H,)), jnp.zeros((H,), f32)], axis=0)

    rkey = jax.random.key(42)
    rnd_cols = [jax.random.uniform(jax.random.fold_in(rkey, i), (NQ, 1))
                for i in range(1, MAX_AGENT + 1)]
    rnd_cols.append(jax.random.uniform(jax.random.fold_in(rkey, 999), (NQ, 1)))
    rnd_cols.append(jnp.zeros((NQ, 1), f32))
    rnd = jnp.concatenate(rnd_cols, axis=1)

    grid = (NQ // BM,)
    row_spec = lambda w: pl.BlockSpec((BM, w), lambda i: (i, 0))
    rep_spec = lambda r, w: pl.BlockSpec((r, w), lambda i: (0, 0))
    w_spec = lambda blk: pl.BlockSpec((D, H), lambda i, _b=blk: (_b, 0))

    out = pl.pallas_call(
        _router_kernel,
        grid=grid,
        in_specs=[
            row_spec(D),
            rep_spec(D, H),
            w_spec(0),
            w_spec(1),
            w_spec(0),
            w_spec(1),
            w_spec(2),
            rep_spec(D, H),
            rep_spec(D, H),
            rep_spec(D, H),
            rep_spec(NT, D),
            rep_spec(16, D),
            rep_spec(H, 8),
            rep_spec(8, H),
            row_spec(4),
        ],
        out_specs=[
            row_spec(NT), row_spec(NL), row_spec(1), row_spec(1), row_spec(1),
        ],
        out_shape=[
            jax.ShapeDtypeStruct((NQ, NT), f32),
            jax.ShapeDtypeStruct((NQ, NL), f32),
            jax.ShapeDtypeStruct((NQ, 1), f32),
            jax.ShapeDtypeStruct((NQ, 1), jnp.int32),
            jax.ShapeDtypeStruct((NQ, 1), f32),
        ],
        scratch_shapes=[
            pltpu.VMEM((D, 3 * H), _BF),
            pltpu.VMEM((NT, H), f32),
            pltpu.VMEM((NL, H), f32),
            pltpu.VMEM((NR, H), f32),
            pltpu.VMEM((3 * NT, 2 * H), _BF),
            pltpu.VMEM((3 * NPP, H), _BF),
            pltpu.VMEM((NL * NL, D), _BF),
        ],
    )(queries, Wq, Wqt, Wqt, Wqtl, Wqtl, Wqtl, Wt, Wl, Wr,
      tasks, lr, wdp, bias, rnd)

    task_scores, selected_llm, llm_log_probs, r_sel, r_log_probs = out
    return (task_scores, selected_llm, llm_log_probs,
            r_sel.reshape(NQ), r_log_probs)
